# baseline (speedup 1.0000x reference)
import functools

import jax
import jax.numpy as jnp
from jax import lax
from jax.experimental import pallas as pl
from jax.experimental.pallas import tpu as pltpu
from jax.experimental.pallas import tpu_sc as plsc

N = 50000
NU = 30000
H = 128
BN = 2000
NT = 16
NPAD = 50048
RPT = NPAD // NT
CH = 32
NCH = 4
EB = 128
E = 400000
EPT = E // NT
NBF = EPT // EB
TAIL = EPT - NBF * EB
R = 4
D = R // 2
NG = 47
DB = 1024


def _sc_body(with_counts, *refs):
    if with_counts:
        (tbl, ps, pd, ns, nd, sump, sumn, cnts,
         src_v, dst_v, rows_v, gidx_v, didx_v, ones_v, zb2, bb, acc,
         *sems) = refs
    else:
        (tbl, ps, pd, ns, nd, sump, sumn,
         src_v, dst_v, rows_v, gidx_v, didx_v, ones_v, zb2, bb, acc,
         *sems) = refs
    cid = lax.axis_index("c")
    sid = lax.axis_index("s")
    gs = sems[:R]
    ss = sems[R:]

    def build(slot, b, off):
        for kk in range(EB // 16):
            sv = src_v[pl.ds(b * EB + 16 * kk, 16)]
            gidx_v[slot, pl.ds(16 * kk, 16)] = sv + off
            dv = dst_v[pl.ds(b * EB + 16 * kk, 16)]
            didx_v[slot, pl.ds(16 * kk, 16)] = dv

    def build_tail(slot, off):
        for kk in range(EB // 16):
            valid = lax.iota(jnp.int32, 16) + (16 * kk) < TAIL
            sv = src_v[pl.ds(NBF * EB + 16 * kk, 16)]
            gidx_v[slot, pl.ds(16 * kk, 16)] = jnp.where(valid, sv + off, 0)
            dv = dst_v[pl.ds(NBF * EB + 16 * kk, 16)]
            didx_v[slot, pl.ds(16 * kk, 16)] = jnp.where(valid, dv, N)

    def build_cnt(slot, b):
        for kk in range(EB // 16):
            dv = dst_v[pl.ds(b * EB + 16 * kk, 16)]
            didx_v[slot, pl.ds(16 * kk, 16)] = dv

    def build_cnt_tail(slot):
        for kk in range(EB // 16):
            valid = lax.iota(jnp.int32, 16) + (16 * kk) < TAIL
            dv = dst_v[pl.ds(NBF * EB + 16 * kk, 16)]
            didx_v[slot, pl.ds(16 * kk, 16)] = jnp.where(valid, dv, N)

    def g_start(slot):
        pltpu.async_copy(tbl.at[gidx_v.at[slot]], rows_v.at[slot], gs[slot])

    def g_wait(slot):
        pltpu.make_async_copy(
            tbl.at[gidx_v.at[slot]], rows_v.at[slot], gs[slot]).wait()

    def s_start(slot):
        pltpu.async_copy(rows_v.at[slot], acc.at[didx_v.at[slot]], ss[slot],
                         add=True)

    def s_wait(slot):
        pltpu.make_async_copy(
            rows_v.at[slot], acc.at[didx_v.at[slot]], ss[slot]).wait()

    def scatter_job(off):
        for b in range(D):
            build(b, b, off)
            g_start(b)

        def body(m, _):
            for j in range(R):
                b = R * m + j
                g_wait(j)
                if j < D:
                    @pl.when(b >= D)
                    def _():
                        s_wait((j + D) % R)
                else:
                    s_wait((j + D) % R)
                s_start(j)
                build((j + D) % R, b + D, off)
                g_start((j + D) % R)
            return 0

        lax.fori_loop(0, NG, body, 0)
        for b in range(NG * R, NBF + 1):
            j = b % R
            g_wait(j)
            s_wait((j + D) % R)
            s_start(j)
            nb = b + D
            if nb == NBF:
                build_tail((j + D) % R, off)
                g_start((j + D) % R)
            elif nb < NBF:
                build((j + D) % R, nb, off)
                g_start((j + D) % R)
        for b in range(NBF + 1 - D, NBF + 1):
            s_wait(b % R)

    def c_start(slot):
        pltpu.async_copy(ones_v, acc.at[didx_v.at[slot]], ss[slot], add=True)

    def c_wait(slot):
        pltpu.make_async_copy(
            ones_v, acc.at[didx_v.at[slot]], ss[slot]).wait()

    def counts_job():
        def body(k, _):
            for j in range(2):
                b = 2 * k + j

                @pl.when(k > 0)
                def _():
                    c_wait(j)

                build_cnt(j, b)
                c_start(j)
            return 0

        lax.fori_loop(0, NBF // 2, body, 0)
        c_wait(0)
        build_cnt(0, NBF - 1)
        c_start(0)
        c_wait(1)
        build_cnt_tail(1)
        c_start(1)
        c_wait(0)
        c_wait(1)

    zrow = jnp.zeros((CH,), jnp.bfloat16)
    orow = jnp.ones((CH,), jnp.bfloat16)

    def fillbody(i, _):
        zb2[i, pl.ds(0, CH)] = zrow
        ones_v[i, pl.ds(0, CH)] = orow
        return 0

    lax.fori_loop(0, EB, fillbody, 0)

    base = sid * RPT

    def zero_acc():
        def zbody(i, _):
            pltpu.sync_copy(zb2, acc.at[pl.ds(base + i * EB, EB)])
            return 0

        lax.fori_loop(0, RPT // EB, zbody, 0)
        pltpu.sync_copy(zb2.at[pl.ds(0, RPT - (RPT // EB) * EB)],
                        acc.at[pl.ds(base + (RPT // EB) * EB,
                                     RPT - (RPT // EB) * EB)])

    def drain_acc(oref, row0):
        def dbody(h, _):
            pltpu.sync_copy(acc.at[pl.ds(base + h * DB, DB)], bb)
            pltpu.sync_copy(bb, oref.at[pl.ds(row0 + base + h * DB, DB)])
            return 0

        lax.fori_loop(0, RPT // DB, dbody, 0)
        rem = RPT - (RPT // DB) * DB
        pltpu.sync_copy(acc.at[pl.ds(base + RPT - rem, rem)],
                        bb.at[pl.ds(0, rem)])
        pltpu.sync_copy(bb.at[pl.ds(0, rem)],
                        oref.at[pl.ds(row0 + base + RPT - rem, rem)])

    for ei, (sref, dref, oref) in enumerate(((ps, pd, sump), (ns, nd, sumn))):
        pltpu.sync_copy(sref.at[pl.ds(sid * EPT, EPT)],
                        src_v.at[pl.ds(0, EPT)])
        pltpu.sync_copy(dref.at[pl.ds(sid * EPT, EPT)],
                        dst_v.at[pl.ds(0, EPT)])

        if with_counts:
            @pl.when(cid == ei)
            def _():
                zero_acc()
                plsc.subcore_barrier()
                counts_job()
                plsc.subcore_barrier()
                drain_acc(cnts, ei * NPAD)
                plsc.subcore_barrier()

        for jj in range(NCH // 2):
            chunk = 2 * jj + cid
            zero_acc()
            plsc.subcore_barrier()
            scatter_job(chunk * N)
            plsc.subcore_barrier()
            drain_acc(oref, chunk * NPAD)
            plsc.subcore_barrier()


def _make_segsum(with_counts):
    out_type = [
        jax.ShapeDtypeStruct((NCH * NPAD, CH), jnp.bfloat16),
        jax.ShapeDtypeStruct((NCH * NPAD, CH), jnp.bfloat16),
    ]
    if with_counts:
        out_type.append(jax.ShapeDtypeStruct((2 * NPAD, CH), jnp.bfloat16))
    return functools.partial(
        pl.kernel,
        out_type=out_type,
        mesh=plsc.VectorSubcoreMesh(core_axis_name="c", subcore_axis_name="s"),
        compiler_params=pltpu.CompilerParams(use_tc_tiling_on_sc=False),
        scratch_types=[
            pltpu.VMEM((EPT + EB, ), jnp.int32),
            pltpu.VMEM((EPT + EB, ), jnp.int32),
            pltpu.VMEM((R, EB, CH), jnp.bfloat16),
            pltpu.VMEM((R, EB), jnp.int32),
            pltpu.VMEM((R, EB), jnp.int32),
            pltpu.VMEM((EB, CH), jnp.bfloat16),
            pltpu.VMEM((EB, CH), jnp.bfloat16),
            pltpu.VMEM((DB, CH), jnp.bfloat16),
            pltpu.VMEM_SHARED((NPAD, CH), jnp.bfloat16),
            pltpu.SemaphoreType.DMA,
            pltpu.SemaphoreType.DMA,
            pltpu.SemaphoreType.DMA,
            pltpu.SemaphoreType.DMA,
            pltpu.SemaphoreType.DMA,
            pltpu.SemaphoreType.DMA,
            pltpu.SemaphoreType.DMA,
            pltpu.SemaphoreType.DMA,
        ],
    )(functools.partial(_sc_body, with_counts))


_segsum_cnt = _make_segsum(True)
_segsum_nc = _make_segsum(False)


def _chunk_body(u_ref, it_ref, x_ref, t_ref):
    i = pl.program_id(0)
    v = jnp.where(i < NU // BN, u_ref[...], it_ref[...])
    x_ref[...] = v
    for c in range(NCH):
        t_ref[c] = v[:, c * CH:(c + 1) * CH].astype(jnp.bfloat16)


def _chunker(users, items):
    return pl.pallas_call(
        _chunk_body,
        grid=(N // BN,),
        in_specs=[
            pl.BlockSpec((BN, H), lambda i: (jnp.minimum(i, NU // BN - 1), 0)),
            pl.BlockSpec((BN, H), lambda i: (jnp.maximum(i - NU // BN, 0), 0)),
        ],
        out_specs=[
            pl.BlockSpec((BN, H), lambda i: (i, 0)),
            pl.BlockSpec((NCH, BN, CH), lambda i: (0, i, 0)),
        ],
        out_shape=[
            jax.ShapeDtypeStruct((N, H), jnp.float32),
            jax.ShapeDtypeStruct((NCH, N, CH), jnp.bfloat16),
        ],
    )(users, items)


def _dense_body(emit_table, spc_ref, snc_ref, x_ref, cp_ref, cn_ref,
                u_ref, v_ref, w_ref, b_ref, *outs):
    sp = jnp.concatenate([spc_ref[c] for c in range(NCH)],
                         axis=-1).astype(jnp.float32)
    sn = jnp.concatenate([snc_ref[c] for c in range(NCH)],
                         axis=-1).astype(jnp.float32)
    rp = 1.0 / jnp.clip(cp_ref[...].astype(jnp.float32), 1.0, None)
    rn = 1.0 / jnp.clip(cn_ref[...].astype(jnp.float32), 1.0, None)
    acc = jnp.dot(sp * rp, u_ref[...], preferred_element_type=jnp.float32)
    acc = acc + jnp.dot(sn * rn, v_ref[...],
                        preferred_element_type=jnp.float32)
    acc = acc + jnp.dot(x_ref[...], w_ref[...],
                        preferred_element_type=jnp.float32)
    z = jnp.maximum(acc + b_ref[...], 0.0)
    outs[0][...] = z
    if emit_table:
        for c in range(NCH):
            outs[1][c] = z[:, c * CH:(c + 1) * CH].astype(jnp.bfloat16)


def _dense_layer(emit_table, spc, snc, x, cp, cn, u, v, w, b):
    bs = pl.BlockSpec((BN, H), lambda i: (i, 0))
    cs = pl.BlockSpec((BN, 1), lambda i: (i, 0))
    ws = pl.BlockSpec((H, H), lambda i: (0, 0))
    sums = pl.BlockSpec((NCH, BN, CH), lambda i: (0, i, 0))
    out_specs = [bs]
    out_shape = [jax.ShapeDtypeStruct((N, H), jnp.float32)]
    if emit_table:
        out_specs.append(pl.BlockSpec((NCH, BN, CH), lambda i: (0, i, 0)))
        out_shape.append(jax.ShapeDtypeStruct((NCH, N, CH), jnp.bfloat16))
    return pl.pallas_call(
        functools.partial(_dense_body, emit_table),
        grid=(N // BN,),
        in_specs=[sums, sums, bs, cs, cs, ws, ws, ws,
                  pl.BlockSpec((1, H), lambda i: (0, 0))],
        out_specs=out_specs,
        out_shape=out_shape,
    )(spc, snc, x, cp, cn, u, v, w, b)


def kernel(users_emb, items_emb, Wpl1, Wpr1, bpr1, Wnl1, Wnr1, bnr1,
           Wpl2, Wpr2, bpr2, Wnl2, Wnr2, bnr2,
           pos_edge_index, neg_edge_index):
    ps = pos_edge_index[0].astype(jnp.int32)
    pd = pos_edge_index[1].astype(jnp.int32)
    ns = neg_edge_index[0].astype(jnp.int32)
    nd = neg_edge_index[1].astype(jnp.int32)

    z64 = jnp.zeros((H // 2, H // 2), jnp.float32)
    u1 = jnp.concatenate([Wpl1, jnp.zeros_like(Wpl1)], axis=1)
    v1 = jnp.concatenate([jnp.zeros_like(Wnl1), Wnl1], axis=1)
    w1 = jnp.concatenate([Wpr1, Wnr1], axis=1)
    b1 = jnp.concatenate([bpr1, bnr1])[None, :]
    u2 = jnp.block([[Wpl2[:64], z64], [z64, Wnl2[:64]]])
    v2 = jnp.block([[z64, Wnl2[64:]], [Wpl2[64:], z64]])
    w2 = jnp.block([[Wpr2, z64], [z64, Wnr2]])
    b2 = jnp.concatenate([bpr2, bnr2])[None, :]

    x, xt = _chunker(users_emb, items_emb)
    sp1f, sn1f, cnts = _segsum_cnt(xt.reshape(NCH * N, CH), ps, pd, ns, nd)
    cp = cnts[:N, :1]
    cn = cnts[NPAD:NPAD + N, :1]
    z, zt = _dense_layer(True, sp1f.reshape(NCH, NPAD, CH),
                         sn1f.reshape(NCH, NPAD, CH), x, cp, cn,
                         u1, v1, w1, b1)

    sp2f, sn2f = _segsum_nc(zt.reshape(NCH * N, CH), ps, pd, ns, nd)
    (out,) = _dense_layer(False, sp2f.reshape(NCH, NPAD, CH),
                          sn2f.reshape(NCH, NPAD, CH), z, cp, cn,
                          u2, v2, w2, b2)
    return out
